# initial kernel scaffold (unmeasured)
import jax
import jax.numpy as jnp
from jax import lax
from jax.experimental import pallas as pl
from jax.experimental.pallas import tpu as pltpu

N_DEV = 4
SQ = 1024
SKV_SHARD = 1024
HQ = 8
DH = 128
D = 1024
SCALE = 0.08838834764831843


def kernel(x, Wq, K_ext, V_ext, Wo):
    def body(x_ref, wq_ref, k_ref, v_ref, wo_ref, out_ref,
             kv_all, q_buf, ctx_buf, send_sems, recv_sems):
        my = lax.axis_index("i")
        left = lax.rem(my + N_DEV - 1, N_DEV)
        right = lax.rem(my + 1, N_DEV)

        barrier = pltpu.get_barrier_semaphore()
        for nbr in (left, right):
            pl.semaphore_signal(barrier, inc=1, device_id=(nbr,),
                                device_id_type=pl.DeviceIdType.MESH)
        pl.semaphore_wait(barrier, 2)

        kv_all[my, 0] = k_ref[0].astype(jnp.bfloat16)
        kv_all[my, 1] = v_ref[0].astype(jnp.bfloat16)

        for hop in range(N_DEV - 1):
            idx = lax.rem(my - hop + N_DEV, N_DEV)
            rdma = pltpu.make_async_remote_copy(
                src_ref=kv_all.at[idx],
                dst_ref=kv_all.at[idx],
                send_sem=send_sems.at[hop],
                recv_sem=recv_sems.at[hop],
                device_id=(right,),
                device_id_type=pl.DeviceIdType.MESH,
            )
            rdma.start()
            if hop == 0:
                q_buf[...] = jnp.dot(
                    x_ref[0].astype(jnp.bfloat16),
                    wq_ref[...].astype(jnp.bfloat16),
                    preferred_element_type=jnp.float32,
                ).astype(jnp.bfloat16)
            rdma.wait()

        qi = lax.broadcasted_iota(jnp.int32, (SQ, SKV_SHARD), 0)
        kj = lax.broadcasted_iota(jnp.int32, (SQ, SKV_SHARD), 1)
        mask = ((qi // 64) % 4) == ((kj // 64) % 4)

        q = q_buf[...]
        for h in range(HQ):
            qh = q[:, h * DH:(h + 1) * DH]
            ss = []
            for c in range(N_DEV):
                kc = kv_all[c, 0, :, h, :]
                s = lax.dot_general(
                    qh, kc, (((1,), (1,)), ((), ())),
                    preferred_element_type=jnp.float32,
                )
                ss.append(jnp.where(mask, s * SCALE, -1e9))
            s_all = jnp.concatenate(ss, axis=1)
            m = jnp.max(s_all, axis=1, keepdims=True)
            p = jnp.exp(s_all - m)
            l = jnp.sum(p, axis=1, keepdims=True)
            pb = (p / l).astype(jnp.bfloat16)
            acc = None
            for c in range(N_DEV):
                vc = kv_all[c, 1, :, h, :]
                t = jnp.dot(
                    pb[:, c * SKV_SHARD:(c + 1) * SKV_SHARD], vc,
                    preferred_element_type=jnp.float32,
                )
                acc = t if acc is None else acc + t
            ctx_buf[:, h * DH:(h + 1) * DH] = acc.astype(jnp.bfloat16)

        out_ref[0] = jnp.dot(
            ctx_buf[...], wo_ref[...].astype(jnp.bfloat16),
            preferred_element_type=jnp.float32,
        )

    return pl.pallas_call(
        body,
        out_shape=jax.ShapeDtypeStruct((1, SQ, D), jnp.float32),
        in_specs=[pl.BlockSpec(memory_space=pltpu.VMEM)] * 5,
        out_specs=pl.BlockSpec(memory_space=pltpu.VMEM),
        scratch_shapes=[
            pltpu.VMEM((N_DEV, 2, SKV_SHARD, HQ, DH), jnp.bfloat16),
            pltpu.VMEM((SQ, D), jnp.bfloat16),
            pltpu.VMEM((SQ, D), jnp.bfloat16),
            pltpu.SemaphoreType.DMA((N_DEV - 1,)),
            pltpu.SemaphoreType.DMA((N_DEV - 1,)),
        ],
        compiler_params=pltpu.CompilerParams(collective_id=0),
    )(x, Wq, K_ext, V_ext, Wo)


# baseline (device time: 277860 ns/iter reference)
import jax
import jax.numpy as jnp
from jax import lax
from jax.experimental import pallas as pl
from jax.experimental.pallas import tpu as pltpu

N_DEV = 4
SQ = 1024
SKV = 1024
HQ = 8
DH = 128
D = 1024
TQ = 128
SCALE = 0.08838834764831843


def kernel(x, Wq, K_ext, V_ext, Wo):
    def body(x_ref, wq_ref, k_ref, v_ref, wo_ref, out_ref,
             kv_all, q_buf, ctx_buf, send_sems, recv_sems):
        my = lax.axis_index("i")
        left = lax.rem(my + N_DEV - 1, N_DEV)
        right = lax.rem(my + 1, N_DEV)

        barrier = pltpu.get_barrier_semaphore()
        for nbr in (left, right):
            pl.semaphore_signal(barrier, inc=1, device_id=(nbr,),
                                device_id_type=pl.DeviceIdType.MESH)
        pl.semaphore_wait(barrier, 2)

        kv_all[my, 0] = k_ref[...].astype(jnp.bfloat16)
        kv_all[my, 1] = v_ref[...].astype(jnp.bfloat16)

        for hop in range(N_DEV - 1):
            idx = lax.rem(my - hop + N_DEV, N_DEV)
            rdma = pltpu.make_async_remote_copy(
                src_ref=kv_all.at[idx],
                dst_ref=kv_all.at[idx],
                send_sem=send_sems.at[hop],
                recv_sem=recv_sems.at[hop],
                device_id=(right,),
                device_id_type=pl.DeviceIdType.MESH,
            )
            rdma.start()
            if hop == 0:
                wq = wq_ref[...].astype(jnp.bfloat16)

                def qproj_tile(qt, _, wq=wq):
                    r = pl.ds(qt * TQ, TQ)
                    q_buf[r, :] = jnp.dot(
                        x_ref[r, :].astype(jnp.bfloat16), wq,
                        preferred_element_type=jnp.float32,
                    ).astype(jnp.bfloat16)
                    return 0

                lax.fori_loop(0, SQ // TQ, qproj_tile, 0)
            rdma.wait()

        qi = lax.broadcasted_iota(jnp.int32, (TQ, SKV), 0)
        kj = lax.broadcasted_iota(jnp.int32, (TQ, SKV), 1)

        def attn_tile(qt, _):
            mask = ((qt * (TQ // 64) + qi // 64) % 4) == ((kj // 64) % 4)
            r = pl.ds(qt * TQ, TQ)
            for h in range(HQ):
                hs = slice(h * DH, (h + 1) * DH)
                qh = q_buf[r, hs]
                m_run = jnp.full((TQ, 1), -1e38, jnp.float32)
                l_run = jnp.zeros((TQ, 1), jnp.float32)
                acc = jnp.zeros((TQ, DH), jnp.float32)
                for c in range(N_DEV):
                    kc = kv_all[c, 0, :, hs]
                    s = lax.dot_general(
                        qh, kc, (((1,), (1,)), ((), ())),
                        preferred_element_type=jnp.float32,
                    )
                    s = jnp.where(mask, s * SCALE, -1e9)
                    m_new = jnp.maximum(
                        m_run, jnp.max(s, axis=1, keepdims=True))
                    corr = jnp.exp(m_run - m_new)
                    p = jnp.exp(s - m_new)
                    l_run = l_run * corr + jnp.sum(p, axis=1, keepdims=True)
                    vc = kv_all[c, 1, :, hs]
                    acc = acc * corr + jnp.dot(
                        p.astype(jnp.bfloat16), vc,
                        preferred_element_type=jnp.float32,
                    )
                    m_run = m_new
                ctx_buf[r, hs] = (acc / l_run).astype(jnp.bfloat16)
            return 0

        lax.fori_loop(0, SQ // TQ, attn_tile, 0)

        wo = wo_ref[...].astype(jnp.bfloat16)

        def out_tile(qt, _):
            r = pl.ds(qt * TQ, TQ)
            out_ref[r, :] = jnp.dot(
                ctx_buf[r, :], wo, preferred_element_type=jnp.float32)
            return 0

        lax.fori_loop(0, SQ // TQ, out_tile, 0)

    out2d = pl.pallas_call(
        body,
        out_shape=jax.ShapeDtypeStruct((SQ, D), jnp.float32),
        in_specs=[pl.BlockSpec(memory_space=pltpu.VMEM)] * 5,
        out_specs=pl.BlockSpec(memory_space=pltpu.VMEM),
        scratch_shapes=[
            pltpu.VMEM((N_DEV, 2, SKV, HQ * DH), jnp.bfloat16),
            pltpu.VMEM((SQ, D), jnp.bfloat16),
            pltpu.VMEM((SQ, D), jnp.bfloat16),
            pltpu.SemaphoreType.DMA((N_DEV - 1,)),
            pltpu.SemaphoreType.DMA((N_DEV - 1,)),
        ],
        compiler_params=pltpu.CompilerParams(
            collective_id=0, vmem_limit_bytes=100 * 1024 * 1024
        ),
    )(
        x.reshape(SQ, D),
        Wq,
        K_ext.reshape(SKV, HQ * DH),
        V_ext.reshape(SKV, HQ * DH),
        Wo,
    )
    return out2d.reshape(1, SQ, D)


# device time: 87500 ns/iter; 3.1755x vs baseline; 3.1755x over previous
import jax
import jax.numpy as jnp
from jax import lax
from jax.experimental import pallas as pl
from jax.experimental.pallas import tpu as pltpu

N_DEV = 4
SQ = 1024
SKV = 1024
HQ = 8
DH = 128
D = 1024
TQ = 128
NB = 16
GS = 256
SCALE = 0.08838834764831843


def kernel(x, Wq, K_ext, V_ext, Wo):
    def body(x_ref, wq_ref, k_ref, v_ref, wo_ref, out_ref,
             comm, stats, kvb, q_buf, ctx_buf,
             o_send, o_recv, s_send, s_recv):
        my = lax.axis_index("i")

        barrier = pltpu.get_barrier_semaphore()
        for d in range(1, N_DEV):
            pl.semaphore_signal(barrier, inc=1,
                                device_id=(lax.rem(my + d, N_DEV),),
                                device_id_type=pl.DeviceIdType.MESH)
        pl.semaphore_wait(barrier, N_DEV - 1)

        for b in range(NB):
            pb = (b % 4) * 4 + b // 4
            src = pl.ds(b * 64, 64)
            dst = pl.ds(pb * 64, 64)
            kvb[0, dst, :] = k_ref[src, :].astype(jnp.bfloat16)
            kvb[1, dst, :] = v_ref[src, :].astype(jnp.bfloat16)

        wq = wq_ref[...].astype(jnp.bfloat16)

        def qproj_tile(t, _, wq=wq):
            p0 = 2 * t
            rows = []
            for dp in range(2):
                borig = (p0 + dp) // 4 + 4 * ((p0 + dp) % 4)
                rows.append(
                    x_ref[pl.ds(borig * 64, 64), :].astype(jnp.bfloat16))
            xt = jnp.concatenate(rows, axis=0)
            q_buf[pl.ds(t * TQ, TQ), :] = jnp.dot(
                xt, wq, preferred_element_type=jnp.float32
            ).astype(jnp.bfloat16)
            return 0

        lax.fori_loop(0, SQ // TQ, qproj_tile, 0)

        for m in range(4):
            g = slice(m * GS, (m + 1) * GS)
            for h in range(HQ):
                hs = slice(h * DH, (h + 1) * DH)
                qg = q_buf[g, hs]
                kc = kvb[0, g, hs]
                s_t = lax.dot_general(
                    kc, qg, (((1,), (1,)), ((), ())),
                    preferred_element_type=jnp.float32,
                ) * SCALE
                p_t = jnp.exp(s_t)
                stats[my, h:h + 1, g] = jnp.sum(p_t, axis=0, keepdims=True)
                vc = kvb[1, g, hs]
                o = lax.dot_general(
                    p_t.astype(jnp.bfloat16), vc, (((0,), (0,)), ((), ())),
                    preferred_element_type=jnp.float32,
                )
                comm[my, g, hs] = o.astype(jnp.bfloat16)

        o_rdmas, s_rdmas = [], []
        for d in range(N_DEV - 1):
            peer = lax.rem(my + 1 + d, N_DEV)
            r = pltpu.make_async_remote_copy(
                src_ref=comm.at[my], dst_ref=comm.at[my],
                send_sem=o_send.at[d], recv_sem=o_recv.at[d],
                device_id=(peer,), device_id_type=pl.DeviceIdType.MESH,
            )
            r.start()
            o_rdmas.append(r)
            r = pltpu.make_async_remote_copy(
                src_ref=stats.at[my], dst_ref=stats.at[my],
                send_sem=s_send.at[d], recv_sem=s_recv.at[d],
                device_id=(peer,), device_id_type=pl.DeviceIdType.MESH,
            )
            r.start()
            s_rdmas.append(r)

        ii = lax.broadcasted_iota(jnp.int32, (SQ, SQ), 0)
        jj = lax.broadcasted_iota(jnp.int32, (SQ, SQ), 1)
        eye = (ii == jj).astype(jnp.bfloat16)

        for r in o_rdmas + s_rdmas:
            r.wait()

        den_sum = (stats[0] + stats[1] + stats[2] + stats[3])
        den_cols = lax.dot_general(
            eye, den_sum.astype(jnp.bfloat16), (((1,), (1,)), ((), ())),
            preferred_element_type=jnp.float32,
        )
        for h in range(HQ):
            hs = slice(h * DH, (h + 1) * DH)
            num = (comm[0, :, hs].astype(jnp.float32)
                   + comm[1, :, hs].astype(jnp.float32)
                   + comm[2, :, hs].astype(jnp.float32)
                   + comm[3, :, hs].astype(jnp.float32))
            ctx_buf[:, hs] = (num / den_cols[:, h:h + 1]).astype(jnp.bfloat16)

        wo = wo_ref[...].astype(jnp.bfloat16)

        def out_tile(t, _, wo=wo):
            rows = []
            for dp in range(2):
                b = 2 * t + dp
                prow = (b % 4) * GS + (b // 4) * 64
                rows.append(ctx_buf[pl.ds(prow, 64), :])
            ct = jnp.concatenate(rows, axis=0)
            out_ref[pl.ds(t * TQ, TQ), :] = jnp.dot(
                ct, wo, preferred_element_type=jnp.float32)
            return 0

        lax.fori_loop(0, SQ // TQ, out_tile, 0)

    out2d = pl.pallas_call(
        body,
        out_shape=jax.ShapeDtypeStruct((SQ, D), jnp.float32),
        in_specs=[pl.BlockSpec(memory_space=pltpu.VMEM)] * 5,
        out_specs=pl.BlockSpec(memory_space=pltpu.VMEM),
        scratch_shapes=[
            pltpu.VMEM((N_DEV, SQ, D), jnp.bfloat16),
            pltpu.VMEM((N_DEV, HQ, SQ), jnp.float32),
            pltpu.VMEM((2, SKV, D), jnp.bfloat16),
            pltpu.VMEM((SQ, D), jnp.bfloat16),
            pltpu.VMEM((SQ, D), jnp.bfloat16),
            pltpu.SemaphoreType.DMA((N_DEV - 1,)),
            pltpu.SemaphoreType.DMA((N_DEV - 1,)),
            pltpu.SemaphoreType.DMA((N_DEV - 1,)),
            pltpu.SemaphoreType.DMA((N_DEV - 1,)),
        ],
        compiler_params=pltpu.CompilerParams(
            collective_id=0, vmem_limit_bytes=100 * 1024 * 1024
        ),
    )(
        x.reshape(SQ, D),
        Wq,
        K_ext.reshape(SKV, HQ * DH),
        V_ext.reshape(SKV, HQ * DH),
        Wo,
    )
    return out2d.reshape(1, SQ, D)


# device time: 61406 ns/iter; 4.5250x vs baseline; 1.4249x over previous
import jax
import jax.numpy as jnp
from jax import lax
from jax.experimental import pallas as pl
from jax.experimental.pallas import tpu as pltpu

N_DEV = 4
SQ = 1024
SKV = 1024
HQ = 8
DH = 128
D = 1024
TQ = 128
NB = 16
GS = 256
SCALE = 0.08838834764831843


def kernel(x, Wq, K_ext, V_ext, Wo):
    def body(x_ref, wq_ref, k_ref, v_ref, wo_ref, out_ref,
             o_part, l_part, scat, sscat, obuf, kvb, q_buf, ctx_q,
             o_send, o_recv, s_send, s_recv, f_send, f_recv):
        my = lax.axis_index("i")

        barrier = pltpu.get_barrier_semaphore()
        for d in range(1, N_DEV):
            pl.semaphore_signal(barrier, inc=1,
                                device_id=(lax.rem(my + d, N_DEV),),
                                device_id_type=pl.DeviceIdType.MESH)
        pl.semaphore_wait(barrier, N_DEV - 1)

        for b in range(NB):
            pb = (b % 4) * 4 + b // 4
            src = pl.ds(b * 64, 64)
            dst = pl.ds(pb * 64, 64)
            kvb[0, dst, :] = k_ref[src, :].astype(jnp.bfloat16)
            kvb[1, dst, :] = v_ref[src, :].astype(jnp.bfloat16)

        wq = wq_ref[...].astype(jnp.bfloat16)

        def qproj_tile(t, _, wq=wq):
            p0 = 2 * t
            rows = []
            for dp in range(2):
                borig = (p0 + dp) // 4 + 4 * ((p0 + dp) % 4)
                rows.append(
                    x_ref[pl.ds(borig * 64, 64), :].astype(jnp.bfloat16))
            xt = jnp.concatenate(rows, axis=0)
            q_buf[pl.ds(t * TQ, TQ), :] = jnp.dot(
                xt, wq, preferred_element_type=jnp.float32
            ).astype(jnp.bfloat16)
            return 0

        lax.fori_loop(0, SQ // TQ, qproj_tile, 0)

        for m in range(4):
            g = slice(m * GS, (m + 1) * GS)
            for h in range(HQ):
                hs = slice(h * DH, (h + 1) * DH)
                qg = q_buf[g, hs]
                kc = kvb[0, g, hs]
                s_t = lax.dot_general(
                    kc, qg, (((1,), (1,)), ((), ())),
                    preferred_element_type=jnp.float32,
                ) * SCALE
                p_t = jnp.exp(s_t)
                l_part[h:h + 1, g] = jnp.sum(p_t, axis=0, keepdims=True)
                vc = kvb[1, g, hs]
                o = lax.dot_general(
                    p_t.astype(jnp.bfloat16), vc, (((0,), (0,)), ((), ())),
                    preferred_element_type=jnp.float32,
                )
                o_part[g, hs] = o.astype(jnp.bfloat16)

        scat[my] = o_part[pl.ds(my * GS, GS), :]
        sscat[my] = l_part[:, pl.ds(my * GS, GS)]
        qtr_rdmas = []
        for d in range(N_DEV - 1):
            peer = lax.rem(my + 1 + d, N_DEV)
            r = pltpu.make_async_remote_copy(
                src_ref=o_part.at[pl.ds(peer * GS, GS), :],
                dst_ref=scat.at[my],
                send_sem=o_send.at[d], recv_sem=o_recv.at[d],
                device_id=(peer,), device_id_type=pl.DeviceIdType.MESH,
            )
            r.start()
            qtr_rdmas.append(r)
            r = pltpu.make_async_remote_copy(
                src_ref=l_part.at[:, pl.ds(peer * GS, GS)],
                dst_ref=sscat.at[my],
                send_sem=s_send.at[d], recv_sem=s_recv.at[d],
                device_id=(peer,), device_id_type=pl.DeviceIdType.MESH,
            )
            r.start()
            qtr_rdmas.append(r)

        ii = lax.broadcasted_iota(jnp.int32, (GS, GS), 0)
        jj = lax.broadcasted_iota(jnp.int32, (GS, GS), 1)
        eye = (ii == jj).astype(jnp.bfloat16)

        for r in qtr_rdmas:
            r.wait()

        den_sum = (sscat[0] + sscat[1] + sscat[2] + sscat[3])
        den_cols = lax.dot_general(
            eye, den_sum.astype(jnp.bfloat16), (((1,), (1,)), ((), ())),
            preferred_element_type=jnp.float32,
        )
        for h in range(HQ):
            hs = slice(h * DH, (h + 1) * DH)
            num = (scat[0, :, hs].astype(jnp.float32)
                   + scat[1, :, hs].astype(jnp.float32)
                   + scat[2, :, hs].astype(jnp.float32)
                   + scat[3, :, hs].astype(jnp.float32))
            ctx_q[:, hs] = (num / den_cols[:, h:h + 1]).astype(jnp.bfloat16)

        wo = wo_ref[...].astype(jnp.bfloat16)
        obuf[my] = jnp.dot(
            ctx_q[...], wo, preferred_element_type=jnp.float32
        ).astype(jnp.bfloat16)

        fin_rdmas = []
        for d in range(N_DEV - 1):
            peer = lax.rem(my + 1 + d, N_DEV)
            r = pltpu.make_async_remote_copy(
                src_ref=obuf.at[my], dst_ref=obuf.at[my],
                send_sem=f_send.at[d], recv_sem=f_recv.at[d],
                device_id=(peer,), device_id_type=pl.DeviceIdType.MESH,
            )
            r.start()
            fin_rdmas.append(r)
        for r in fin_rdmas:
            r.wait()

        for c in range(N_DEV):
            for j in range(4):
                out_ref[pl.ds((c + 4 * j) * 64, 64), :] = (
                    obuf[c, pl.ds(j * 64, 64), :].astype(jnp.float32))

    out2d = pl.pallas_call(
        body,
        out_shape=jax.ShapeDtypeStruct((SQ, D), jnp.float32),
        in_specs=[pl.BlockSpec(memory_space=pltpu.VMEM)] * 5,
        out_specs=pl.BlockSpec(memory_space=pltpu.VMEM),
        scratch_shapes=[
            pltpu.VMEM((SQ, D), jnp.bfloat16),
            pltpu.VMEM((HQ, SQ), jnp.float32),
            pltpu.VMEM((N_DEV, GS, D), jnp.bfloat16),
            pltpu.VMEM((N_DEV, HQ, GS), jnp.float32),
            pltpu.VMEM((N_DEV, GS, D), jnp.bfloat16),
            pltpu.VMEM((2, SKV, D), jnp.bfloat16),
            pltpu.VMEM((SQ, D), jnp.bfloat16),
            pltpu.VMEM((GS, D), jnp.bfloat16),
            pltpu.SemaphoreType.DMA((N_DEV - 1,)),
            pltpu.SemaphoreType.DMA((N_DEV - 1,)),
            pltpu.SemaphoreType.DMA((N_DEV - 1,)),
            pltpu.SemaphoreType.DMA((N_DEV - 1,)),
            pltpu.SemaphoreType.DMA((N_DEV - 1,)),
            pltpu.SemaphoreType.DMA((N_DEV - 1,)),
        ],
        compiler_params=pltpu.CompilerParams(
            collective_id=0, vmem_limit_bytes=100 * 1024 * 1024
        ),
    )(
        x.reshape(SQ, D),
        Wq,
        K_ext.reshape(SKV, HQ * DH),
        V_ext.reshape(SKV, HQ * DH),
        Wo,
    )
    return out2d.reshape(1, SQ, D)


# device time: 54094 ns/iter; 5.1366x vs baseline; 1.1352x over previous
import jax
import jax.numpy as jnp
from jax import lax
from jax.experimental import pallas as pl
from jax.experimental.pallas import tpu as pltpu

N_DEV = 4
SQ = 1024
SKV = 1024
HQ = 8
DH = 128
D = 1024
GS = 256
SCALE = 0.08838834764831843


def kernel(x, Wq, K_ext, V_ext, Wo):
    def body(x_ref, wq_ref, k_ref, v_ref, wo_ref, out_ref,
             o_part, l_part, scat, sscat, obuf, ctx_q,
             o_send, o_recv, s_send, s_recv, f_send, f_recv):
        my = lax.axis_index("i")

        barrier = pltpu.get_barrier_semaphore()
        for d in range(1, N_DEV):
            pl.semaphore_signal(barrier, inc=1,
                                device_id=(lax.rem(my + d, N_DEV),),
                                device_id_type=pl.DeviceIdType.MESH)
        pl.semaphore_wait(barrier, N_DEV - 1)

        wq = (wq_ref[...] * SCALE).astype(jnp.bfloat16)

        send_rdmas = []
        for mm in (1, 2, 3, 0):
            m = lax.rem(my + mm, N_DEV)
            xg = jnp.concatenate(
                [x_ref[pl.ds((m + 4 * j) * 64, 64), :] for j in range(4)],
                axis=0).astype(jnp.bfloat16)
            qv = jnp.dot(xg, wq,
                         preferred_element_type=jnp.float32
                         ).astype(jnp.bfloat16)
            for h in range(HQ):
                hs = slice(h * DH, (h + 1) * DH)
                qg = qv[:, hs]
                kc = k_ref[pl.ds(m * GS, GS), hs]
                s_t = lax.dot_general(
                    kc, qg, (((1,), (1,)), ((), ())),
                    preferred_element_type=jnp.float32,
                )
                p_t = jnp.exp(s_t)
                l_part[h:h + 1, pl.ds(m * GS, GS)] = (
                    jnp.sum(p_t, axis=0, keepdims=True))
                vc = v_ref[pl.ds(m * GS, GS), hs]
                o = lax.dot_general(
                    p_t.astype(jnp.bfloat16), vc, (((0,), (0,)), ((), ())),
                    preferred_element_type=jnp.float32,
                )
                o_part[pl.ds(m * GS, GS), hs] = o.astype(jnp.bfloat16)
            if mm != 0:
                d = mm - 1
                r = pltpu.make_async_remote_copy(
                    src_ref=o_part.at[pl.ds(m * GS, GS), :],
                    dst_ref=scat.at[my],
                    send_sem=o_send.at[d], recv_sem=o_recv.at[d],
                    device_id=(m,), device_id_type=pl.DeviceIdType.MESH,
                )
                r.start()
                send_rdmas.append(r)
                r = pltpu.make_async_remote_copy(
                    src_ref=l_part.at[:, pl.ds(m * GS, GS)],
                    dst_ref=sscat.at[my],
                    send_sem=s_send.at[d], recv_sem=s_recv.at[d],
                    device_id=(m,), device_id_type=pl.DeviceIdType.MESH,
                )
                r.start()
                send_rdmas.append(r)

        scat[my] = o_part[pl.ds(my * GS, GS), :]
        sscat[my] = l_part[:, pl.ds(my * GS, GS)]

        ii = lax.broadcasted_iota(jnp.int32, (GS, GS), 0)
        jj = lax.broadcasted_iota(jnp.int32, (GS, GS), 1)
        eye = (ii == jj).astype(jnp.bfloat16)

        for r in send_rdmas:
            r.wait()

        den_sum = (sscat[0] + sscat[1] + sscat[2] + sscat[3])
        den_cols = lax.dot_general(
            eye, den_sum.astype(jnp.bfloat16), (((1,), (1,)), ((), ())),
            preferred_element_type=jnp.float32,
        )
        for h in range(HQ):
            hs = slice(h * DH, (h + 1) * DH)
            num = (scat[0, :, hs].astype(jnp.float32)
                   + scat[1, :, hs].astype(jnp.float32)
                   + scat[2, :, hs].astype(jnp.float32)
                   + scat[3, :, hs].astype(jnp.float32))
            ctx_q[:, hs] = (num / den_cols[:, h:h + 1]).astype(jnp.bfloat16)

        wo = wo_ref[...].astype(jnp.bfloat16)
        obuf[my] = jnp.dot(
            ctx_q[...], wo, preferred_element_type=jnp.float32
        ).astype(jnp.bfloat16)

        fin = []
        for d in range(N_DEV - 1):
            peer = lax.rem(my + 1 + d, N_DEV)
            r = pltpu.make_async_remote_copy(
                src_ref=obuf.at[my], dst_ref=obuf.at[my],
                send_sem=f_send.at[d], recv_sem=f_recv.at[d],
                device_id=(peer,), device_id_type=pl.DeviceIdType.MESH,
            )
            r.start()
            fin.append(r)
        for j in range(4):
            out_ref[pl.ds((my + 4 * j) * 64, 64), :] = (
                obuf[my, pl.ds(j * 64, 64), :].astype(jnp.float32))
        for d in range(N_DEV - 1):
            fin[d].wait()
            s = lax.rem(my + 3 - d, N_DEV)
            for j in range(4):
                out_ref[pl.ds((s + 4 * j) * 64, 64), :] = (
                    obuf[s, pl.ds(j * 64, 64), :].astype(jnp.float32))

    def perm_cast(a):
        return (a.reshape(4, 4, 64, HQ * DH)
                .transpose(1, 0, 2, 3)
                .reshape(SKV, HQ * DH)
                .astype(jnp.bfloat16))

    out2d = pl.pallas_call(
        body,
        out_shape=jax.ShapeDtypeStruct((SQ, D), jnp.float32),
        in_specs=[pl.BlockSpec(memory_space=pltpu.VMEM)] * 5,
        out_specs=pl.BlockSpec(memory_space=pltpu.VMEM),
        scratch_shapes=[
            pltpu.VMEM((SQ, D), jnp.bfloat16),
            pltpu.VMEM((HQ, SQ), jnp.float32),
            pltpu.VMEM((N_DEV, GS, D), jnp.bfloat16),
            pltpu.VMEM((N_DEV, HQ, GS), jnp.float32),
            pltpu.VMEM((N_DEV, GS, D), jnp.bfloat16),
            pltpu.VMEM((GS, D), jnp.bfloat16),
            pltpu.SemaphoreType.DMA((N_DEV - 1,)),
            pltpu.SemaphoreType.DMA((N_DEV - 1,)),
            pltpu.SemaphoreType.DMA((N_DEV - 1,)),
            pltpu.SemaphoreType.DMA((N_DEV - 1,)),
            pltpu.SemaphoreType.DMA((N_DEV - 1,)),
            pltpu.SemaphoreType.DMA((N_DEV - 1,)),
        ],
        compiler_params=pltpu.CompilerParams(
            collective_id=0, vmem_limit_bytes=100 * 1024 * 1024
        ),
    )(
        x.reshape(SQ, D),
        Wq,
        perm_cast(K_ext),
        perm_cast(V_ext),
        Wo,
    )
    return out2d.reshape(1, SQ, D)
